# baseline (speedup 1.0000x reference)
import functools

import jax
import jax.numpy as jnp
from jax import lax
from jax.experimental import pallas as pl
from jax.experimental.pallas import tpu as pltpu
from jax.experimental.pallas import tpu_sc as plsc

_N = 1_000_000
_K = 8192
_NPAD = 1 << 20
_NT = 16
_CHUNK = _NPAD // _NT
_VI = _CHUNK // 16
_CAP = 1024
_WCAP = 2560
_INT_MIN = -2147483648
_LEVELS = 3

_mesh = plsc.VectorSubcoreMesh(core_axis_name="c", subcore_axis_name="s",
                               num_cores=1)


_BITS = (10, 8, 8)
_SHIFTS = (22, 14, 6)
_NBINS = (1024, 256, 256)
_SH_OFF = (0, 16384, 20480)


@functools.partial(
    pl.kernel,
    out_type=(jax.ShapeDtypeStruct((_NT * _CAP,), jnp.int32),
              jax.ShapeDtypeStruct((_NT * _CAP,), jnp.int32)),
    mesh=_mesh,
    compiler_params=pltpu.CompilerParams(use_tc_tiling_on_sc=False,
                                         needs_layout_passes=False),
    scratch_types=[
        pltpu.VMEM((_CHUNK,), jnp.int32),
        pltpu.VMEM((16 * 1024,), jnp.int32),
        pltpu.VMEM((16 * 1024,), jnp.int32),
        pltpu.VMEM((1024,), jnp.int32),
        pltpu.VMEM((1024,), jnp.int32),
        pltpu.VMEM((_CAP,), jnp.int32),
        pltpu.VMEM((_CAP,), jnp.int32),
        pltpu.VMEM((_WCAP,), jnp.int32),
        pltpu.VMEM((_WCAP,), jnp.int32),
        pltpu.VMEM_SHARED((24576,), jnp.int32),
    ],
)
def _sc_select(keys_hbm, ck_hbm, ci_hbm,
               keys_v, hist_v, rb_v, loc_v, tot_v, ck_v, ci_v,
               wk_v, wi_v, sh_hist):
    wid = lax.axis_index("s")
    lanes = lax.iota(jnp.int32, 16)
    ones16 = jnp.ones((16,), jnp.int32)
    zeros16 = jnp.zeros((16,), jnp.int32)
    intmin16 = jnp.full((16,), _INT_MIN, jnp.int32)

    def _zero_hist(j, _):
        hist_v[pl.ds(j * 16, 16)] = zeros16
        return 0

    lax.fori_loop(0, _NBINS[0], _zero_hist, 0)

    pltpu.sync_copy(keys_hbm.at[pl.ds(wid * _CHUNK, _CHUNK)], keys_v)

    idx_base = wid * _CHUNK

    above = jnp.int32(0)
    prefix = jnp.int32(0)
    for lv in range(_LEVELS):
        nb = _NBINS[lv]
        sh = _SHIFTS[lv]
        lanebase = lanes * nb

        if lv == 0:
            def _scan0(j, _, lanebase=lanebase):
                for u in range(4):
                    kv = keys_v[pl.ds(j * 64 + u * 16, 16)]
                    digit = ((kv >> 22) & 0x3FF) ^ 0x200
                    plsc.addupdate_scatter(hist_v, [lanebase + digit], ones16)
                return 0

            lax.fori_loop(0, _VI // 4, _scan0, 0)
        else:
            def _scanw(j, _, sh=sh, prefix=prefix, lanebase=lanebase):
                kv = wk_v[pl.ds(j * 16, 16)]
                digit = (kv >> sh) & 0xFF
                m = (kv >> (sh + 8)) == prefix
                plsc.addupdate_scatter(hist_v, [lanebase + digit], ones16,
                                       mask=m)
                return 0

            lax.fori_loop(0, _WCAP // 16, _scanw, 0)

        def _lred(j, _, nb=nb):
            def _acc(l, acc, j=j):
                return acc + hist_v[pl.ds(l * nb + j * 16, 16)]

            loc_v[pl.ds(j * 16, 16)] = lax.fori_loop(0, 16, _acc, zeros16)
            return 0

        lax.fori_loop(0, nb // 16, _lred, 0)

        pltpu.sync_copy(loc_v.at[pl.ds(0, nb)],
                        sh_hist.at[pl.ds(_SH_OFF[lv] + wid * nb, nb)])
        plsc.subcore_barrier()
        pltpu.sync_copy(sh_hist.at[pl.ds(_SH_OFF[lv], 16 * nb)],
                        rb_v.at[pl.ds(0, 16 * nb)])

        def _tot(j, carry, nb=nb, above=above):
            c_hi, truecnt = carry
            j_rev = nb // 16 - 1 - j

            def _acc(l, acc, j_rev=j_rev):
                return acc + rb_v[pl.ds(l * nb + j_rev * 16, 16)]

            tot = lax.fori_loop(0, 16, _acc, zeros16)
            tot_v[pl.ds(j_rev * 16, 16)] = tot
            t_sum = jnp.sum(tot)
            cum = plsc.cumsum(tot)
            ge_cnt = c_hi + (t_sum - cum) + tot
            cond = (above + ge_cnt) >= _K
            truecnt = truecnt + jnp.sum(cond.astype(jnp.int32))
            return (c_hi + t_sum, truecnt)

        _, truecnt = lax.fori_loop(0, nb // 16, _tot,
                                   (jnp.int32(0), jnp.int32(0)))
        crit = truecnt - 1

        def _above(j, acc, crit=crit):
            dv = j * 16 + lanes
            tot = tot_v[pl.ds(j * 16, 16)]
            return acc + jnp.sum(jnp.where(dv > crit, tot, 0))

        above = above + lax.fori_loop(0, nb // 16, _above, jnp.int32(0))
        if lv == 0:
            prefix = crit - 512
        else:
            prefix = prefix * 256 + crit

        if lv != _LEVELS - 1:
            lax.fori_loop(0, _NBINS[lv + 1], _zero_hist, 0)

        if lv == 0:
            wlo = prefix * (1 << 22)

            def _zero_w(j, _):
                wk_v[pl.ds(j * 16, 16)] = intmin16
                wi_v[pl.ds(j * 16, 16)] = zeros16
                return 0

            lax.fori_loop(0, _WCAP // 16, _zero_w, 0)

            def _wcompact(j, cnt, wlo=wlo):
                for u in range(2):
                    kv = keys_v[pl.ds(j * 32 + u * 16, 16)]
                    m = kv >= wlo
                    c = jnp.minimum(cnt, _WCAP - 16)
                    plsc.store_compressed(wk_v.at[pl.ds(c, 16)], kv, mask=m)
                    iv = idx_base + j * 32 + u * 16 + lanes
                    plsc.store_compressed(wi_v.at[pl.ds(c, 16)], iv, mask=m)
                    cnt = cnt + plsc.all_reduce_population_count(m)[0]
                return cnt

            lax.fori_loop(0, _VI // 2, _wcompact, jnp.int32(0))

    thresh = prefix * 64

    def _zero_cand(j, _):
        ck_v[pl.ds(j * 16, 16)] = intmin16
        ci_v[pl.ds(j * 16, 16)] = zeros16
        return 0

    lax.fori_loop(0, _CAP // 16, _zero_cand, 0)

    def _compact(j, cnt):
        kv = wk_v[pl.ds(j * 16, 16)]
        m = kv >= thresh
        c = jnp.minimum(cnt, _CAP - 16)
        plsc.store_compressed(ck_v.at[pl.ds(c, 16)], kv, mask=m)
        iv = wi_v[pl.ds(j * 16, 16)]
        plsc.store_compressed(ci_v.at[pl.ds(c, 16)], iv, mask=m)
        return cnt + jnp.sum(m.astype(jnp.int32))

    lax.fori_loop(0, _WCAP // 16, _compact, jnp.int32(0))

    pltpu.sync_copy(ck_v, ck_hbm.at[pl.ds(wid * _CAP, _CAP)])
    pltpu.sync_copy(ci_v, ci_hbm.at[pl.ds(wid * _CAP, _CAP)])


def _log2(v):
    return v.bit_length() - 1


def _roll(x, s, axis):
    return pltpu.roll(x, s % x.shape[axis], axis)


def _sort_body(k_ref, i_ref, o_ref):
    kk = k_ref[...]
    ii = i_ref[...]
    lin = (lax.broadcasted_iota(jnp.int32, (128, 128), 0) * 128
           + lax.broadcasted_iota(jnp.int32, (128, 128), 1))
    n = 128 * 128
    size = 2
    while size <= n:
        d = size // 2
        while d >= 1:
            if d >= 128:
                ax, m = 0, d // 128
            else:
                ax, m = 1, d
            hb = (lin & d) != 0
            pk = jnp.where(hb, _roll(kk, m, ax), _roll(kk, -m, ax))
            pi = jnp.where(hb, _roll(ii, m, ax), _roll(ii, -m, ax))
            same = (((lin >> _log2(size)) ^ (lin >> _log2(d))) & 1) == 0
            eq = kk == pk
            lt_pm = (pk > kk) | (eq & (pi < ii))
            lt_mp = (kk > pk) | (eq & (ii < pi))
            take = (same & lt_pm) | ((~same) & lt_mp)
            kk = jnp.where(take, pk, kk)
            ii = jnp.where(take, pi, ii)
            d //= 2
        size *= 2
    o_ref[...] = lax.slice(ii, (0, 0), (64, 128))


_sort_tc = pl.pallas_call(
    _sort_body,
    out_shape=jax.ShapeDtypeStruct((64, 128), jnp.int32),
)


def kernel(item_id, sample_distribution):
    g = jax.random.gumbel(jax.random.key(42), (_N,), dtype=jnp.float32)
    scores = jnp.log(sample_distribution) + g
    b = lax.bitcast_convert_type(scores, jnp.int32)
    keys = b ^ ((b >> 31) & 0x7FFFFFFF)
    keys = jnp.concatenate(
        [keys, jnp.full((_NPAD - _N,), _INT_MIN, jnp.int32)])
    ck, ci = _sc_select(keys)
    neg = _sort_tc(ck.reshape(128, 128), ci.reshape(128, 128))
    return (item_id, neg.reshape(_K))


# speedup vs baseline: 1.1649x; 1.1649x over previous
import functools

import jax
import jax.numpy as jnp
from jax import lax
from jax.experimental import pallas as pl
from jax.experimental.pallas import tpu as pltpu
from jax.experimental.pallas import tpu_sc as plsc

_N = 1_000_000
_K = 8192
_NPAD = 1 << 20
_NT = 16
_CHUNK = _NPAD // _NT
_VI = _CHUNK // 16
_CAP = 1024
_WCAP = 2560
_INT_MIN = -2147483648
_LEVELS = 3

_mesh = plsc.VectorSubcoreMesh(core_axis_name="c", subcore_axis_name="s",
                               num_cores=1)


_BITS = (10, 8, 8)
_SHIFTS = (22, 14, 6)
_NBINS = (1024, 256, 256)
_SH_OFF = (0, 16384, 20480)


@functools.partial(
    pl.kernel,
    out_type=(jax.ShapeDtypeStruct((_NT * _CAP,), jnp.int32),
              jax.ShapeDtypeStruct((_NT * _CAP,), jnp.int32)),
    mesh=_mesh,
    compiler_params=pltpu.CompilerParams(use_tc_tiling_on_sc=False,
                                         needs_layout_passes=False),
    scratch_types=[
        pltpu.VMEM((_CHUNK,), jnp.int32),
        pltpu.VMEM((16 * 1026,), jnp.int32),
        pltpu.VMEM((16 * 1024,), jnp.int32),
        pltpu.VMEM((1024,), jnp.int32),
        pltpu.VMEM((1024,), jnp.int32),
        pltpu.VMEM((_CAP,), jnp.int32),
        pltpu.VMEM((_CAP,), jnp.int32),
        pltpu.VMEM((_WCAP,), jnp.int32),
        pltpu.VMEM((_WCAP,), jnp.int32),
        pltpu.VMEM_SHARED((24576,), jnp.int32),
    ],
)
def _sc_select(keys_hbm, ck_hbm, ci_hbm,
               keys_v, hist_v, rb_v, loc_v, tot_v, ck_v, ci_v,
               wk_v, wi_v, sh_hist):
    wid = lax.axis_index("s")
    lanes = lax.iota(jnp.int32, 16)
    ones16 = jnp.ones((16,), jnp.int32)
    zeros16 = jnp.zeros((16,), jnp.int32)
    intmin16 = jnp.full((16,), _INT_MIN, jnp.int32)

    def _zero_hist(j, _):
        hist_v[pl.ds(j * 16, 16)] = zeros16
        return 0

    lax.fori_loop(0, _NBINS[0] + 1, _zero_hist, 0)

    pltpu.sync_copy(keys_hbm.at[pl.ds(wid * _CHUNK, _CHUNK)], keys_v)

    idx_base = wid * _CHUNK

    above = jnp.int32(0)
    prefix = jnp.int32(0)
    for lv in range(_LEVELS):
        nb = _NBINS[lv]
        sh = _SHIFTS[lv]
        lstride = nb + 1
        lanebase = lanes * lstride

        if lv == 0:
            def _scan0(j, _, lanebase=lanebase):
                for u in range(4):
                    kv = keys_v[pl.ds(j * 64 + u * 16, 16)]
                    digit = ((kv >> 22) & 0x3FF) ^ 0x200
                    plsc.addupdate_scatter(hist_v, [lanebase + digit], ones16)
                return 0

            lax.fori_loop(0, _VI // 4, _scan0, 0)
        else:
            def _scanw(j, _, sh=sh, prefix=prefix, lanebase=lanebase):
                kv = wk_v[pl.ds(j * 16, 16)]
                digit = (kv >> sh) & 0xFF
                m = (kv >> (sh + 8)) == prefix
                plsc.addupdate_scatter(hist_v, [lanebase + digit], ones16,
                                       mask=m)
                return 0

            lax.fori_loop(0, _WCAP // 16, _scanw, 0)

        def _lred(j, _, nb=nb):
            def _acc(l, acc, j=j, nb=nb):
                return acc + hist_v[pl.ds(l * (nb + 1) + j * 16, 16)]

            loc_v[pl.ds(j * 16, 16)] = lax.fori_loop(0, 16, _acc, zeros16)
            return 0

        lax.fori_loop(0, nb // 16, _lred, 0)

        pltpu.sync_copy(loc_v.at[pl.ds(0, nb)],
                        sh_hist.at[pl.ds(_SH_OFF[lv] + wid * nb, nb)])
        plsc.subcore_barrier()
        pltpu.sync_copy(sh_hist.at[pl.ds(_SH_OFF[lv], 16 * nb)],
                        rb_v.at[pl.ds(0, 16 * nb)])

        def _tot(j, carry, nb=nb, above=above):
            c_hi, truecnt = carry
            j_rev = nb // 16 - 1 - j

            def _acc(l, acc, j_rev=j_rev):
                return acc + rb_v[pl.ds(l * nb + j_rev * 16, 16)]

            tot = lax.fori_loop(0, 16, _acc, zeros16)
            tot_v[pl.ds(j_rev * 16, 16)] = tot
            t_sum = jnp.sum(tot)
            cum = plsc.cumsum(tot)
            ge_cnt = c_hi + (t_sum - cum) + tot
            cond = (above + ge_cnt) >= _K
            truecnt = truecnt + jnp.sum(cond.astype(jnp.int32))
            return (c_hi + t_sum, truecnt)

        _, truecnt = lax.fori_loop(0, nb // 16, _tot,
                                   (jnp.int32(0), jnp.int32(0)))
        crit = truecnt - 1

        def _above(j, acc, crit=crit):
            dv = j * 16 + lanes
            tot = tot_v[pl.ds(j * 16, 16)]
            return acc + jnp.sum(jnp.where(dv > crit, tot, 0))

        above = above + lax.fori_loop(0, nb // 16, _above, jnp.int32(0))
        if lv == 0:
            prefix = crit - 512
        else:
            prefix = prefix * 256 + crit

        if lv != _LEVELS - 1:
            lax.fori_loop(0, _NBINS[lv + 1] + 1, _zero_hist, 0)

        if lv == 0:
            wlo = prefix * (1 << 22)

            def _zero_w(j, _):
                wk_v[pl.ds(j * 16, 16)] = intmin16
                wi_v[pl.ds(j * 16, 16)] = zeros16
                return 0

            lax.fori_loop(0, _WCAP // 16, _zero_w, 0)

            def _wcompact(j, cnt, wlo=wlo):
                for u in range(2):
                    kv = keys_v[pl.ds(j * 32 + u * 16, 16)]
                    m = kv >= wlo
                    c = jnp.minimum(cnt, _WCAP - 16)
                    plsc.store_compressed(wk_v.at[pl.ds(c, 16)], kv, mask=m)
                    iv = idx_base + j * 32 + u * 16 + lanes
                    plsc.store_compressed(wi_v.at[pl.ds(c, 16)], iv, mask=m)
                    cnt = cnt + plsc.all_reduce_population_count(m)[0]
                return cnt

            lax.fori_loop(0, _VI // 2, _wcompact, jnp.int32(0))

    thresh = prefix * 64

    def _zero_cand(j, _):
        ck_v[pl.ds(j * 16, 16)] = intmin16
        ci_v[pl.ds(j * 16, 16)] = zeros16
        return 0

    lax.fori_loop(0, _CAP // 16, _zero_cand, 0)

    def _compact(j, cnt):
        kv = wk_v[pl.ds(j * 16, 16)]
        m = kv >= thresh
        c = jnp.minimum(cnt, _CAP - 16)
        plsc.store_compressed(ck_v.at[pl.ds(c, 16)], kv, mask=m)
        iv = wi_v[pl.ds(j * 16, 16)]
        plsc.store_compressed(ci_v.at[pl.ds(c, 16)], iv, mask=m)
        return cnt + jnp.sum(m.astype(jnp.int32))

    lax.fori_loop(0, _WCAP // 16, _compact, jnp.int32(0))

    pltpu.sync_copy(ck_v, ck_hbm.at[pl.ds(wid * _CAP, _CAP)])
    pltpu.sync_copy(ci_v, ci_hbm.at[pl.ds(wid * _CAP, _CAP)])


def _log2(v):
    return v.bit_length() - 1


def _roll(x, s, axis):
    return pltpu.roll(x, s % x.shape[axis], axis)


def _sort_body(k_ref, i_ref, o_ref):
    kk = k_ref[...]
    ii = i_ref[...]
    lin = (lax.broadcasted_iota(jnp.int32, (128, 128), 0) * 128
           + lax.broadcasted_iota(jnp.int32, (128, 128), 1))
    n = 128 * 128
    size = 2
    while size <= n:
        d = size // 2
        while d >= 1:
            if d >= 128:
                ax, m = 0, d // 128
            else:
                ax, m = 1, d
            hb = (lin & d) != 0
            pk = jnp.where(hb, _roll(kk, m, ax), _roll(kk, -m, ax))
            pi = jnp.where(hb, _roll(ii, m, ax), _roll(ii, -m, ax))
            same = (((lin >> _log2(size)) ^ (lin >> _log2(d))) & 1) == 0
            eq = kk == pk
            lt_pm = (pk > kk) | (eq & (pi < ii))
            lt_mp = (kk > pk) | (eq & (ii < pi))
            take = (same & lt_pm) | ((~same) & lt_mp)
            kk = jnp.where(take, pk, kk)
            ii = jnp.where(take, pi, ii)
            d //= 2
        size *= 2
    o_ref[...] = lax.slice(ii, (0, 0), (64, 128))


_sort_tc = pl.pallas_call(
    _sort_body,
    out_shape=jax.ShapeDtypeStruct((64, 128), jnp.int32),
)


def kernel(item_id, sample_distribution):
    g = jax.random.gumbel(jax.random.key(42), (_N,), dtype=jnp.float32)
    scores = jnp.log(sample_distribution) + g
    b = lax.bitcast_convert_type(scores, jnp.int32)
    keys = b ^ ((b >> 31) & 0x7FFFFFFF)
    keys = jnp.concatenate(
        [keys, jnp.full((_NPAD - _N,), _INT_MIN, jnp.int32)])
    ck, ci = _sc_select(keys)
    neg = _sort_tc(ck.reshape(128, 128), ci.reshape(128, 128))
    return (item_id, neg.reshape(_K))
